# baseline (speedup 1.0000x reference)
import jax
import jax.numpy as jnp
from jax.experimental import pallas as pl
from jax.experimental.pallas import tpu as pltpu

_TOPK = 64
_I32_MAX = 0x7FFFFFFF


def _sortable_key(pre):
    bi = jax.lax.bitcast_convert_type(pre, jnp.int32)
    return jnp.where(bi >= 0, bi, bi ^ jnp.int32(_I32_MAX))


def _mm1_kernel(x_ref, e_ref, b_ref, o_ref):
    o_ref[...] = (
        jnp.dot(x_ref[...], e_ref[...], preferred_element_type=jnp.float32)
        + b_ref[...]
    )


def _bsearch(count_gt, lo0, hi0, iters=32):
    def body(_, lohi):
        lo, hi = lohi
        mid = (lo >> 1) + (hi >> 1) + (lo & hi & 1)
        le = count_gt(mid) <= (_TOPK - 1)
        return jnp.where(le, lo, mid + 1), jnp.where(le, mid, hi)

    _, hi = jax.lax.fori_loop(0, iters, body, (lo0, hi0))
    return hi


def _key_to_float(t):
    return jax.lax.bitcast_convert_type(
        jnp.where(t >= 0, t, t ^ jnp.int32(_I32_MAX)), jnp.float32
    )


def _tkey_kernel(pre_ref, t_ref):
    rows = pre_ref.shape[0]
    lo0 = jnp.full((rows, 1), -1, jnp.int32)
    hi0 = jnp.full((rows, 1), _I32_MAX, jnp.int32)


    sub = pre_ref[:, :1024]

    def cnt_sub(mid):
        return jnp.sum((sub > _key_to_float(mid)).astype(jnp.int32),
                       axis=1, keepdims=True)

    tsub = _bsearch(cnt_sub, lo0, hi0)
    pre = pre_ref[...]
    rmax = _sortable_key(jnp.max(pre, axis=1, keepdims=True))

    lo_i = tsub - 1
    hi_i = jnp.maximum(rmax, tsub)
    clo0 = jnp.sum((pre > _key_to_float(lo_i)).astype(jnp.int32),
                   axis=1, keepdims=True)

    def cond(state):
        lo, hi, _, it = state
        return jnp.any(lo < hi) & (it < 64)

    def body(state):
        lo, hi, clo, it = state
        bis = (lo >> 1) + (hi >> 1) + (lo & hi & 1)
        llo = jnp.log(clo.astype(jnp.float32) + 1.0)
        frac = (llo - 4.1667) / llo
        vlo = _key_to_float(lo)
        vhi = _key_to_float(hi)
        itp = _sortable_key(vlo + frac * (vhi - vlo))
        mid = jnp.where(it % 2 == 0, bis, jnp.clip(itp, lo, jnp.maximum(hi - 1, lo)))
        mid = jnp.where(lo < hi, mid, lo)
        cnt = jnp.sum((pre > _key_to_float(mid)).astype(jnp.int32),
                      axis=1, keepdims=True)
        le = cnt <= (_TOPK - 1)
        lo2 = jnp.where(le, lo, mid + 1)
        hi2 = jnp.where(le, mid, hi)
        clo2 = jnp.where(le, clo, cnt)
        return lo2, hi2, clo2, it + 1

    lo_f, hi_f, _, _ = jax.lax.while_loop(
        cond, body, (lo_i, hi_i, clo0, jnp.int32(0))
    )
    t_ref[...] = hi_f


def _decode_kernel(pre_ref, t_ref, d_ref, b_ref, o_ref):
    kstep = pl.program_id(1)
    pre = pre_ref[...]
    keep = (pre >= _key_to_float(t_ref[...])) & (pre > 0)
    acts = jnp.where(keep, pre, 0.0)
    part = jnp.dot(acts, d_ref[...], preferred_element_type=jnp.float32)

    @pl.when(kstep == 0)
    def _():
        o_ref[...] = jnp.broadcast_to(b_ref[...], o_ref.shape)

    o_ref[...] += part


def kernel(x, encoder, encoder_bias, decoder, decoder_bias):
    m, d_model = x.shape
    d_hidden = encoder.shape[1]

    bm1, bn1 = 512, 2048
    pre = pl.pallas_call(
        _mm1_kernel,
        grid=(d_hidden // bn1, m // bm1),
        in_specs=[
            pl.BlockSpec((bm1, d_model), lambda j, i: (i, 0)),
            pl.BlockSpec((d_model, bn1), lambda j, i: (0, j)),
            pl.BlockSpec((1, bn1), lambda j, i: (0, j)),
        ],
        out_specs=pl.BlockSpec((bm1, bn1), lambda j, i: (i, j)),
        out_shape=jax.ShapeDtypeStruct((m, d_hidden), jnp.float32),
        compiler_params=pltpu.CompilerParams(
            dimension_semantics=("parallel", "parallel")
        ),
    )(x, encoder, encoder_bias.reshape(1, -1))

    bmt = 256
    tkey = pl.pallas_call(
        _tkey_kernel,
        grid=(m // bmt,),
        in_specs=[pl.BlockSpec((bmt, d_hidden), lambda i: (i, 0))],
        out_specs=pl.BlockSpec((bmt, 1), lambda i: (i, 0)),
        out_shape=jax.ShapeDtypeStruct((m, 1), jnp.int32),
        compiler_params=pltpu.CompilerParams(
            dimension_semantics=("parallel",)
        ),
    )(pre)

    bm2, bk2 = 1024, 1024
    recon = pl.pallas_call(
        _decode_kernel,
        grid=(m // bm2, d_hidden // bk2),
        in_specs=[
            pl.BlockSpec((bm2, bk2), lambda i, k: (i, k)),
            pl.BlockSpec((bm2, 1), lambda i, k: (i, 0)),
            pl.BlockSpec((bk2, d_model), lambda i, k: (k, 0)),
            pl.BlockSpec((1, d_model), lambda i, k: (0, 0)),
        ],
        out_specs=pl.BlockSpec((bm2, d_model), lambda i, k: (i, 0)),
        out_shape=jax.ShapeDtypeStruct((m, d_model), jnp.float32),
        compiler_params=pltpu.CompilerParams(
            dimension_semantics=("parallel", "arbitrary")
        ),
    )(pre, tkey, decoder, decoder_bias.reshape(1, -1))

    return recon


# speedup vs baseline: 1.4645x; 1.4645x over previous
import jax
import jax.numpy as jnp
from jax.experimental import pallas as pl
from jax.experimental.pallas import tpu as pltpu

_TOPK = 64
_I32_MAX = 0x7FFFFFFF


def _sortable_key(pre):
    bi = jax.lax.bitcast_convert_type(pre, jnp.int32)
    return jnp.where(bi >= 0, bi, bi ^ jnp.int32(_I32_MAX))


def _mm1_kernel(x_ref, e_ref, b_ref, o_ref):
    o_ref[...] = (
        jnp.dot(x_ref[...], e_ref[...], preferred_element_type=jnp.float32)
        + b_ref[...]
    )


def _bsearch(count_gt, lo0, hi0, iters=32):
    def body(_, lohi):
        lo, hi = lohi
        mid = (lo >> 1) + (hi >> 1) + (lo & hi & 1)
        le = count_gt(mid) <= (_TOPK - 1)
        return jnp.where(le, lo, mid + 1), jnp.where(le, mid, hi)

    _, hi = jax.lax.fori_loop(0, iters, body, (lo0, hi0))
    return hi


def _key_to_float(t):
    return jax.lax.bitcast_convert_type(
        jnp.where(t >= 0, t, t ^ jnp.int32(_I32_MAX)), jnp.float32
    )


def _tkey_kernel(pre_ref, t_ref):
    rows = pre_ref.shape[0]
    lo0 = jnp.full((rows, 1), -1, jnp.int32)
    hi0 = jnp.full((rows, 1), _I32_MAX, jnp.int32)


    sub = pre_ref[:, :1024]

    def cnt_sub(mid):
        return jnp.sum((sub > _key_to_float(mid)).astype(jnp.int32),
                       axis=1, keepdims=True)

    tsub = _bsearch(cnt_sub, lo0, hi0)
    pre = pre_ref[...]
    rmax = _sortable_key(jnp.max(pre, axis=1, keepdims=True))

    def cond(lohi):
        lo, hi = lohi
        return jnp.any(lo < hi)

    def body(lohi):
        lo, hi = lohi
        mid = (lo >> 1) + (hi >> 1) + (lo & hi & 1)
        cnt = jnp.sum((pre > _key_to_float(mid)).astype(jnp.int32),
                      axis=1, keepdims=True)
        le = cnt <= (_TOPK - 1)
        return jnp.where(le, lo, mid + 1), jnp.where(le, mid, hi)

    _, hi = jax.lax.while_loop(cond, body, (tsub - 1, jnp.maximum(rmax, tsub)))
    t_ref[...] = hi


def _decode_kernel(pre_ref, t_ref, d_ref, b_ref, o_ref):
    kstep = pl.program_id(1)
    pre = pre_ref[...]
    keep = (pre >= _key_to_float(t_ref[...])) & (pre > 0)
    acts = jnp.where(keep, pre, 0.0)
    part = jnp.dot(acts, d_ref[...], preferred_element_type=jnp.float32)

    @pl.when(kstep == 0)
    def _():
        o_ref[...] = jnp.broadcast_to(b_ref[...], o_ref.shape)

    o_ref[...] += part


def kernel(x, encoder, encoder_bias, decoder, decoder_bias):
    m, d_model = x.shape
    d_hidden = encoder.shape[1]

    bm1, bn1 = 512, 2048
    pre = pl.pallas_call(
        _mm1_kernel,
        grid=(d_hidden // bn1, m // bm1),
        in_specs=[
            pl.BlockSpec((bm1, d_model), lambda j, i: (i, 0)),
            pl.BlockSpec((d_model, bn1), lambda j, i: (0, j)),
            pl.BlockSpec((1, bn1), lambda j, i: (0, j)),
        ],
        out_specs=pl.BlockSpec((bm1, bn1), lambda j, i: (i, j)),
        out_shape=jax.ShapeDtypeStruct((m, d_hidden), jnp.float32),
        compiler_params=pltpu.CompilerParams(
            dimension_semantics=("parallel", "parallel")
        ),
    )(x, encoder, encoder_bias.reshape(1, -1))

    bmt = 256
    tkey = pl.pallas_call(
        _tkey_kernel,
        grid=(m // bmt,),
        in_specs=[pl.BlockSpec((bmt, d_hidden), lambda i: (i, 0))],
        out_specs=pl.BlockSpec((bmt, 1), lambda i: (i, 0)),
        out_shape=jax.ShapeDtypeStruct((m, 1), jnp.int32),
        compiler_params=pltpu.CompilerParams(
            dimension_semantics=("parallel",)
        ),
    )(pre)

    bm2, bk2 = 1024, 1024
    recon = pl.pallas_call(
        _decode_kernel,
        grid=(m // bm2, d_hidden // bk2),
        in_specs=[
            pl.BlockSpec((bm2, bk2), lambda i, k: (i, k)),
            pl.BlockSpec((bm2, 1), lambda i, k: (i, 0)),
            pl.BlockSpec((bk2, d_model), lambda i, k: (k, 0)),
            pl.BlockSpec((1, d_model), lambda i, k: (0, 0)),
        ],
        out_specs=pl.BlockSpec((bm2, d_model), lambda i, k: (i, 0)),
        out_shape=jax.ShapeDtypeStruct((m, d_model), jnp.float32),
        compiler_params=pltpu.CompilerParams(
            dimension_semantics=("parallel", "arbitrary")
        ),
    )(pre, tkey, decoder, decoder_bias.reshape(1, -1))

    return recon
